# baseline (speedup 1.0000x reference)
import functools

import jax
import jax.numpy as jnp
from jax import lax
from jax.experimental import pallas as pl
from jax.experimental.pallas import tpu as pltpu
from jax.experimental.pallas import tpu_sc as plsc

B = 4096
MAXLEN = 50
A = 128
HNL = 2048
NC = 2
NS = 16
NW = NC * NS
BPW = B // NW
CHUNK = 128
CPW = B * MAXLEN // (CHUNK * NW)


def _sc_gather_pool(tag2d, seg2d, rating, category, zeros, emb_rating,
                    emb_category, emb_tag):
    mesh = plsc.VectorSubcoreMesh(core_axis_name="c", subcore_axis_name="s")
    f32 = jnp.float32

    @functools.partial(
        pl.kernel,
        out_type=(
            jax.ShapeDtypeStruct((B, A), f32),
            jax.ShapeDtypeStruct((B, A), f32),
            jax.ShapeDtypeStruct((B, A), f32),
        ),
        mesh=mesh,
        scratch_types=[
            pltpu.VMEM((CPW, CHUNK), jnp.int32),
            pltpu.VMEM((CPW, CHUNK), jnp.int32),
            pltpu.VMEM((BPW,), jnp.int32),
            pltpu.VMEM((BPW,), jnp.int32),
            pltpu.VMEM((CHUNK, A), f32),
            pltpu.VMEM((CHUNK, A), f32),
            pltpu.VMEM((CHUNK, A), f32),
            pltpu.VMEM((CHUNK, A), f32),
            pltpu.VMEM((CHUNK, A), f32),
            pltpu.VMEM((CHUNK, A), f32),
            pltpu.VMEM_SHARED((B // NC, A), f32),
            pltpu.SemaphoreType.DMA,
            pltpu.SemaphoreType.DMA,
            pltpu.SemaphoreType.DMA,
            pltpu.SemaphoreType.DMA,
            pltpu.SemaphoreType.DMA,
            pltpu.SemaphoreType.DMA,
        ],
    )
    def body(tag_ref, seg_ref, rat_ref, cat_ref, zero_ref,
             er_ref, ec_ref, et_ref,
             rat_out, cat_out, sum_out,
             idx_v, seg_v, ridx_v, cidx_v, rows0_v, rows1_v, rows2_v,
             rows3_v, rows4_v, rows5_v, acc,
             sem0, sem1, sem2, sem3, sem4, sem5):
        wid = lax.axis_index("c") * NS + lax.axis_index("s")
        base = wid * BPW
        lbase = lax.axis_index("s") * BPW

        pltpu.sync_copy(tag_ref.at[wid], idx_v)

        bufs = (rows0_v, rows1_v, rows2_v, rows3_v, rows4_v, rows5_v)
        sems = (sem0, sem1, sem2, sem3, sem4, sem5)
        nbuf = 6

        for b in range(nbuf):
            pltpu.async_copy(et_ref.at[idx_v.at[b]], bufs[b], sems[b])

        pltpu.sync_copy(zero_ref, acc.at[pl.ds(lbase, BPW)])
        pltpu.sync_copy(seg_ref.at[wid], seg_v)
        pltpu.sync_copy(rat_ref.at[pl.ds(base, BPW)], ridx_v)
        pltpu.sync_copy(cat_ref.at[pl.ds(base, BPW)], cidx_v)

        nfull = (CPW - nbuf) // nbuf
        rem = CPW - nbuf - nfull * nbuf

        def drain(c, b):
            pltpu.make_async_copy(et_ref.at[idx_v.at[c]], bufs[b],
                                  sems[b]).wait()
            pltpu.sync_copy(bufs[b], acc.at[seg_v.at[c]], add=True)

        def step(t, carry):
            for b in range(nbuf):
                c = nbuf * t + b
                drain(c, b)
                pltpu.async_copy(et_ref.at[idx_v.at[c + nbuf]], bufs[b],
                                 sems[b])
            return carry

        lax.fori_loop(0, nfull, step, 0)

        cbase = nfull * nbuf
        for b in range(nbuf):
            c = cbase + b
            drain(c, b)
            if b < rem:
                pltpu.async_copy(et_ref.at[idx_v.at[c + nbuf]], bufs[b],
                                 sems[b])
        for b in range(rem):
            drain(cbase + nbuf + b, b)

        rd = pltpu.async_copy(er_ref.at[ridx_v], bufs[0], sems[0])
        cd = pltpu.async_copy(ec_ref.at[cidx_v], bufs[1], sems[1])
        pltpu.sync_copy(acc.at[pl.ds(lbase, BPW)],
                        sum_out.at[pl.ds(base, BPW)])
        rd.wait()
        pltpu.sync_copy(bufs[0], rat_out.at[pl.ds(base, BPW)])
        cd.wait()
        pltpu.sync_copy(bufs[1], cat_out.at[pl.ds(base, BPW)])

    return body(tag2d, seg2d, rating, category, zeros, emb_rating,
                emb_category, emb_tag)


def _tc_project(rat, cat, tsum, tag, w, b2d):
    BM = 512

    def body(tag_ref, rat_ref, cat_ref, tsum_ref, w_ref, b_ref,
             attr_ref, enc_ref):
        tl = jnp.sum((tag_ref[...] != 0).astype(jnp.float32), axis=1,
                     keepdims=True)
        tmean = tsum_ref[...] / tl
        x = jnp.concatenate([rat_ref[...], cat_ref[...], tmean], axis=1)
        attr_ref[0] = rat_ref[...]
        attr_ref[1] = cat_ref[...]
        attr_ref[2] = tmean
        y = lax.dot_general(x, w_ref[...], (((1,), (1,)), ((), ())),
                            preferred_element_type=jnp.float32)
        enc_ref[...] = jnp.tanh(y + b_ref[...])[:, None, :]

    return pl.pallas_call(
        body,
        grid=(B // BM,),
        in_specs=[
            pl.BlockSpec((BM, MAXLEN), lambda i: (i, 0)),
            pl.BlockSpec((BM, A), lambda i: (i, 0)),
            pl.BlockSpec((BM, A), lambda i: (i, 0)),
            pl.BlockSpec((BM, A), lambda i: (i, 0)),
            pl.BlockSpec((HNL, 3 * A), lambda i: (0, 0)),
            pl.BlockSpec((1, HNL), lambda i: (0, 0)),
        ],
        out_specs=[
            pl.BlockSpec((3, BM, A), lambda i: (0, i, 0)),
            pl.BlockSpec((BM, 1, HNL), lambda i: (i, 0, 0)),
        ],
        out_shape=[
            jax.ShapeDtypeStruct((3, B, A), jnp.float32),
            jax.ShapeDtypeStruct((B, 1, HNL), jnp.float32),
        ],
    )(tag, rat, cat, tsum, w, b2d)


def kernel(rating, category, tag, emb_rating, emb_category, emb_tag, W_out,
           b_out):
    rating_f = rating.reshape(B).astype(jnp.int32)
    category_f = category.reshape(B).astype(jnp.int32)
    tag_i = tag.astype(jnp.int32)
    tag2d = tag_i.reshape(NW, CPW, CHUNK)
    seg2d = (jnp.repeat(jnp.arange(B, dtype=jnp.int32), MAXLEN)
             % (B // NC)).reshape(NW, CPW, CHUNK)
    zeros = jnp.zeros((BPW, A), jnp.float32)
    rat_e, cat_e, tsum = _sc_gather_pool(tag2d, seg2d, rating_f, category_f,
                                         zeros, emb_rating, emb_category,
                                         emb_tag)
    attr3, enc = _tc_project(rat_e, cat_e, tsum, tag_i, W_out,
                             b_out.reshape(1, HNL))
    return attr3.transpose(1, 0, 2), enc


# speedup vs baseline: 1.0506x; 1.0506x over previous
import functools

import jax
import jax.numpy as jnp
from jax import lax
from jax.experimental import pallas as pl
from jax.experimental.pallas import tpu as pltpu
from jax.experimental.pallas import tpu_sc as plsc

B = 4096
MAXLEN = 50
A = 128
HNL = 2048
NC = 2
NS = 16
NW = NC * NS
BPW = B // NW
CHUNK = 128
CPW = B * MAXLEN // (CHUNK * NW)


def _sc_gather_pool(tag2d, seg2d, rating, category, zeros, emb_rating,
                    emb_category, emb_tag):
    mesh = plsc.VectorSubcoreMesh(core_axis_name="c", subcore_axis_name="s")
    f32 = jnp.float32

    @functools.partial(
        pl.kernel,
        out_type=(
            jax.ShapeDtypeStruct((B, A), f32),
            jax.ShapeDtypeStruct((B, A), f32),
            jax.ShapeDtypeStruct((B, A), f32),
        ),
        mesh=mesh,
        scratch_types=[
            pltpu.VMEM((CPW, CHUNK), jnp.int32),
            pltpu.VMEM((CPW, CHUNK), jnp.int32),
            pltpu.VMEM((BPW,), jnp.int32),
            pltpu.VMEM((BPW,), jnp.int32),
            pltpu.VMEM((CHUNK, A), f32),
            pltpu.VMEM((CHUNK, A), f32),
            pltpu.VMEM((CHUNK, A), f32),
            pltpu.VMEM((CHUNK, A), f32),
            pltpu.VMEM((BPW, A), f32),
            pltpu.VMEM((BPW, A), f32),
            pltpu.VMEM_SHARED((B // NC, A), f32),
            pltpu.SemaphoreType.DMA,
            pltpu.SemaphoreType.DMA,
            pltpu.SemaphoreType.DMA,
            pltpu.SemaphoreType.DMA,
            pltpu.SemaphoreType.DMA,
            pltpu.SemaphoreType.DMA,
        ],
    )
    def body(tag_ref, seg_ref, rat_ref, cat_ref, zero_ref,
             er_ref, ec_ref, et_ref,
             rat_out, cat_out, sum_out,
             idx_v, seg_v, ridx_v, cidx_v, rows0_v, rows1_v, rows2_v,
             rows3_v, remb_v, cemb_v, acc,
             sem0, sem1, sem2, sem3, semr, semc):
        wid = lax.axis_index("c") * NS + lax.axis_index("s")
        base = wid * BPW
        lbase = lax.axis_index("s") * BPW

        pltpu.sync_copy(tag_ref.at[wid], idx_v)

        bufs = (rows0_v, rows1_v, rows2_v, rows3_v)
        sems = (sem0, sem1, sem2, sem3)
        nbuf = 4

        for b in range(nbuf):
            pltpu.async_copy(et_ref.at[idx_v.at[b]], bufs[b], sems[b])

        pltpu.sync_copy(zero_ref, acc.at[pl.ds(lbase, BPW)])
        pltpu.sync_copy(seg_ref.at[wid], seg_v)
        pltpu.sync_copy(rat_ref.at[pl.ds(base, BPW)], ridx_v)
        pltpu.sync_copy(cat_ref.at[pl.ds(base, BPW)], cidx_v)
        rd = pltpu.async_copy(er_ref.at[ridx_v], remb_v, semr)
        cd = pltpu.async_copy(ec_ref.at[cidx_v], cemb_v, semc)

        nfull = (CPW - nbuf) // nbuf
        rem = CPW - nbuf - nfull * nbuf

        def drain(c, b):
            pltpu.make_async_copy(et_ref.at[idx_v.at[c]], bufs[b],
                                  sems[b]).wait()
            pltpu.sync_copy(bufs[b], acc.at[seg_v.at[c]], add=True)

        def step(t, carry):
            for b in range(nbuf):
                c = nbuf * t + b
                drain(c, b)
                pltpu.async_copy(et_ref.at[idx_v.at[c + nbuf]], bufs[b],
                                 sems[b])
            return carry

        lax.fori_loop(0, nfull, step, 0)

        cbase = nfull * nbuf
        for b in range(nbuf):
            c = cbase + b
            drain(c, b)
            if b < rem:
                pltpu.async_copy(et_ref.at[idx_v.at[c + nbuf]], bufs[b],
                                 sems[b])
        for b in range(rem):
            drain(cbase + nbuf + b, b)

        p3 = pltpu.async_copy(acc.at[pl.ds(lbase, BPW)],
                              sum_out.at[pl.ds(base, BPW)], sem0)
        rd.wait()
        p1 = pltpu.async_copy(remb_v, rat_out.at[pl.ds(base, BPW)], semr)
        cd.wait()
        p2 = pltpu.async_copy(cemb_v, cat_out.at[pl.ds(base, BPW)], semc)
        p1.wait()
        p2.wait()
        p3.wait()

    return body(tag2d, seg2d, rating, category, zeros, emb_rating,
                emb_category, emb_tag)


def _tc_project(rat, cat, tsum, tag, w, b2d):
    BM = 512

    def body(tag_ref, rat_ref, cat_ref, tsum_ref, w_ref, b_ref,
             attr_ref, enc_ref):
        tl = jnp.sum((tag_ref[...] != 0).astype(jnp.float32), axis=1,
                     keepdims=True)
        tmean = tsum_ref[...] / tl
        x = jnp.concatenate([rat_ref[...], cat_ref[...], tmean], axis=1)
        attr_ref[0] = rat_ref[...]
        attr_ref[1] = cat_ref[...]
        attr_ref[2] = tmean
        y = lax.dot_general(x, w_ref[...], (((1,), (1,)), ((), ())),
                            preferred_element_type=jnp.float32)
        enc_ref[...] = jnp.tanh(y + b_ref[...])[:, None, :]

    return pl.pallas_call(
        body,
        grid=(B // BM,),
        in_specs=[
            pl.BlockSpec((BM, MAXLEN), lambda i: (i, 0)),
            pl.BlockSpec((BM, A), lambda i: (i, 0)),
            pl.BlockSpec((BM, A), lambda i: (i, 0)),
            pl.BlockSpec((BM, A), lambda i: (i, 0)),
            pl.BlockSpec((HNL, 3 * A), lambda i: (0, 0)),
            pl.BlockSpec((1, HNL), lambda i: (0, 0)),
        ],
        out_specs=[
            pl.BlockSpec((3, BM, A), lambda i: (0, i, 0)),
            pl.BlockSpec((BM, 1, HNL), lambda i: (i, 0, 0)),
        ],
        out_shape=[
            jax.ShapeDtypeStruct((3, B, A), jnp.float32),
            jax.ShapeDtypeStruct((B, 1, HNL), jnp.float32),
        ],
    )(tag, rat, cat, tsum, w, b2d)


def kernel(rating, category, tag, emb_rating, emb_category, emb_tag, W_out,
           b_out):
    rating_f = rating.reshape(B).astype(jnp.int32)
    category_f = category.reshape(B).astype(jnp.int32)
    tag_i = tag.astype(jnp.int32)
    tag2d = tag_i.reshape(NW, CPW, CHUNK)
    seg2d = (jnp.repeat(jnp.arange(B, dtype=jnp.int32), MAXLEN)
             % (B // NC)).reshape(NW, CPW, CHUNK)
    zeros = jnp.zeros((BPW, A), jnp.float32)
    rat_e, cat_e, tsum = _sc_gather_pool(tag2d, seg2d, rating_f, category_f,
                                         zeros, emb_rating, emb_category,
                                         emb_tag)
    attr3, enc = _tc_project(rat_e, cat_e, tsum, tag_i, W_out,
                             b_out.reshape(1, HNL))
    return attr3.transpose(1, 0, 2), enc
